# baseline (speedup 1.0000x reference)
import jax
import jax.numpy as jnp
from jax import lax
from jax.experimental import pallas as pl
from jax.experimental.pallas import tpu as pltpu
from jax.experimental.pallas import tpu_sc as plsc

B, C, N, K = 4, 64, 8192, 16
O = 64
EPS = 1e-5
SLOPE = 0.2
BN = B * N

NC, NS = 2, 16
NW = NC * NS
CPT = BN // NW
CHUNK = 32
NCHUNK = CPT // CHUNK
RPC = CHUNK * K
GPC = RPC // 128
IDX_ROWS = CPT * K // 128


def _mm_body(x_ref, wa_ref, wq_ref, u_ref, v_ref):
    xb = x_ref[0]
    dn = (((0,), (0,)), ((), ()))
    u = lax.dot_general(xb, wa_ref[...], dn,
                        preferred_element_type=jnp.float32)
    v = lax.dot_general(xb, wq_ref[...], dn,
                        preferred_element_type=jnp.float32)
    ub = u.astype(jnp.bfloat16)
    u_ref[...] = jnp.concatenate([ub[:N // 2], ub[N // 2:]], axis=1)
    v_ref[...] = jnp.concatenate([v[:N // 2], v[N // 2:]], axis=1)


def _sc_body(table_hbm, fidx_hbm, m_hbm, s_hbm, s2_hbm,
             idx_all, rows0, rows1, outm, outs, s2acc, sem0, sem1):
    wid = lax.axis_index("s") * NC + lax.axis_index("c")

    pltpu.sync_copy(fidx_hbm.at[pl.ds(wid * IDX_ROWS, IDX_ROWS)], idx_all)

    off = (wid // (NW // B)) * N

    def add_off(r, carry):
        for g in range(8):
            sl = pl.ds(g * 16, 16)
            n = idx_all[r, sl]
            idx_all[r, sl] = (off + 2 * (n & (N // 2 - 1))
                              + lax.shift_right_logical(n, 12))
        return carry
    lax.fori_loop(0, IDX_ROWS, add_off, 0)

    zero = jnp.zeros((16,), jnp.float32)
    for g in range(O // 16):
        s2acc[0, pl.ds(g * 16, 16)] = zero

    def issue(chunk, rows_ref, sem):
        for g in range(GPC):
            pltpu.async_copy(
                table_hbm.at[idx_all.at[chunk * GPC + g]],
                rows_ref.at[pl.ds(g * 128, 128)],
                sem)

    def drain(rows_ref, sem):
        pltpu.make_async_copy(table_hbm.at[pl.ds(0, RPC)], rows_ref, sem).wait()

    def compute(chunk, rows_ref):
        mask_hi = jnp.full((16,), -65536, jnp.int32)

        def bf16x2(w):
            a = lax.bitcast_convert_type(lax.shift_left(w, 16), jnp.float32)
            b = lax.bitcast_convert_type(w & mask_hi, jnp.float32)
            return a, b

        def center_body(c, carry):
            r0 = c * K
            for g2 in range(O // 32):
                col2 = pl.ds(g2 * 16, 16)
                va, vb = bf16x2(rows_ref[r0, col2])
                ma, sa, qa = va, va, va * va
                mb, sb, qb = vb, vb, vb * vb
                for r in range(1, K):
                    va, vb = bf16x2(rows_ref[r0 + r, col2])
                    ma = jnp.maximum(ma, va)
                    sa = sa + va
                    qa = qa + va * va
                    mb = jnp.maximum(mb, vb)
                    sb = sb + vb
                    qb = qb + vb * vb
                cola = pl.ds(g2 * 32, 16)
                colb = pl.ds(g2 * 32 + 16, 16)
                outm[c, cola] = ma
                outm[c, colb] = mb
                outs[c, cola] = sa
                outs[c, colb] = sb
                s2acc[0, cola] = s2acc[0, cola] + qa
                s2acc[0, colb] = s2acc[0, colb] + qb
            return carry
        lax.fori_loop(0, CHUNK, center_body, 0)
        rowb = ((wid // (NW // B)) * (N // 2) + (wid % (NW // B // 2)) * CPT
                + chunk * CHUNK)
        half = pl.ds(((wid % (NW // B)) // (NW // B // 2)) * O, O)
        pltpu.sync_copy(outm, m_hbm.at[pl.ds(rowb, CHUNK), half])
        pltpu.sync_copy(outs, s_hbm.at[pl.ds(rowb, CHUNK), half])

    issue(0, rows0, sem0)

    def outer(p, carry):
        a = p * 2
        drain(rows0, sem0)
        issue(a + 1, rows1, sem1)
        compute(a, rows0)
        drain(rows1, sem1)

        @pl.when(p < NCHUNK // 2 - 1)
        def _():
            issue(a + 2, rows0, sem0)

        compute(a + 1, rows1)
        return carry

    lax.fori_loop(0, NCHUNK // 2, outer, 0)
    pltpu.sync_copy(s2acc.at[0],
                    s2_hbm.at[wid // 2, pl.ds((wid % 2) * O, O)])


def _make_sc_gather():
    return pl.kernel(
        _sc_body,
        out_type=(
            jax.ShapeDtypeStruct((BN // 2, 2 * O), jnp.float32),
            jax.ShapeDtypeStruct((BN // 2, 2 * O), jnp.float32),
            jax.ShapeDtypeStruct((NW // 2, 2 * O), jnp.float32),
        ),
        mesh=plsc.VectorSubcoreMesh(core_axis_name="c", subcore_axis_name="s",
                                    num_cores=NC, num_subcores=NS),
        scratch_types=[
            pltpu.VMEM((IDX_ROWS, 128), jnp.int32),
            pltpu.VMEM((RPC, O // 2), jnp.int32),
            pltpu.VMEM((RPC, O // 2), jnp.int32),
            pltpu.VMEM((CHUNK, O), jnp.float32),
            pltpu.VMEM((CHUNK, O), jnp.float32),
            pltpu.VMEM((1, O), jnp.float32),
            pltpu.SemaphoreType.DMA,
            pltpu.SemaphoreType.DMA,
        ],
        compiler_params=pltpu.CompilerParams(use_tc_tiling_on_sc=False),
    )


def _norm_body(m_ref, s_ref, v_ref, s2_ref, o_ref):
    Sb = s_ref[...]
    vb = v_ref[...]
    Mb = m_ref[...]

    def fold(z):
        return z[:O] + z[O:]

    sS = fold(jnp.sum(Sb, axis=0))
    sv = fold(jnp.sum(vb, axis=0))
    sv2 = fold(jnp.sum(vb * vb, axis=0))
    cross = fold(jnp.sum(vb * Sb, axis=0))
    s2 = fold(jnp.sum(s2_ref[0], axis=0))
    cnt = float(N * K)
    mean = (sS + K * sv) / cnt
    eh2 = (s2 + 2.0 * cross + K * sv2) / cnt
    var = eh2 - mean * mean
    rstd = lax.rsqrt(var + EPS)
    mean2 = jnp.concatenate([mean, mean])
    rstd2 = jnp.concatenate([rstd, rstd])
    t = (Mb + vb - mean2[None, :]) * rstd2[None, :]
    t = jnp.where(t >= 0, t, SLOPE * t)
    o_ref[0] = jnp.concatenate([t[:, :O].T, t[:, O:].T], axis=1)


def kernel(x, idx, W0):
    x = x.astype(jnp.float32)
    idx32 = idx.astype(jnp.int32)
    A = W0[:, :C]
    Qm = W0[:, C:] - A
    perm = [b + (i // 2) + 16 * (i % 2) for b in (0, 32) for i in range(32)]
    wa = A.T[:, perm]
    wq = Qm.T

    u, v = pl.pallas_call(
        _mm_body,
        grid=(B,),
        in_specs=[
            pl.BlockSpec((1, C, N), lambda b: (b, 0, 0)),
            pl.BlockSpec((C, O), lambda b: (0, 0)),
            pl.BlockSpec((C, O), lambda b: (0, 0)),
        ],
        out_specs=[
            pl.BlockSpec((N // 2, 2 * O), lambda b: (b, 0)),
            pl.BlockSpec((N // 2, 2 * O), lambda b: (b, 0)),
        ],
        out_shape=[
            jax.ShapeDtypeStruct((BN // 2, 2 * O), jnp.bfloat16),
            jax.ShapeDtypeStruct((BN // 2, 2 * O), jnp.float32),
        ],
    )(x, wa, wq)

    fidx = idx32.reshape(BN * K // 128, 128)
    table = lax.bitcast_convert_type(
        u.reshape(BN // 2, O, 2), jnp.int32).reshape(BN, O // 2)

    Mg, Sg, S2p = _make_sc_gather()(table, fidx)

    out = pl.pallas_call(
        _norm_body,
        grid=(B,),
        in_specs=[
            pl.BlockSpec((N // 2, 2 * O), lambda b: (b, 0)),
            pl.BlockSpec((N // 2, 2 * O), lambda b: (b, 0)),
            pl.BlockSpec((N // 2, 2 * O), lambda b: (b, 0)),
            pl.BlockSpec((1, NW // B // 2, 2 * O), lambda b: (b, 0, 0)),
        ],
        out_specs=pl.BlockSpec((1, O, N), lambda b: (b, 0, 0)),
        out_shape=jax.ShapeDtypeStruct((B, O, N), jnp.float32),
    )(Mg, Sg, v, S2p.reshape(B, NW // B // 2, 2 * O))
    return out


# speedup vs baseline: 1.0405x; 1.0405x over previous
import jax
import jax.numpy as jnp
from jax import lax
from jax.experimental import pallas as pl
from jax.experimental.pallas import tpu as pltpu
from jax.experimental.pallas import tpu_sc as plsc

B, C, N, K = 4, 64, 8192, 16
O = 64
EPS = 1e-5
SLOPE = 0.2
BN = B * N

NC, NS = 2, 16
NW = NC * NS
CPT = BN // NW
CHUNK = 32
NCHUNK = CPT // CHUNK
RPC = CHUNK * K
GPC = RPC // 128
IDX_ROWS = CPT * K // 128


def _mm_body(x_ref, wa_ref, wq_ref, u_ref, v_ref):
    xb = x_ref[0]
    dn = (((0,), (0,)), ((), ()))
    u = lax.dot_general(xb, wa_ref[...], dn,
                        preferred_element_type=jnp.float32)
    v = lax.dot_general(xb, wq_ref[...], dn,
                        preferred_element_type=jnp.float32)
    ub = u.astype(jnp.bfloat16)
    u_ref[...] = jnp.concatenate([ub[:N // 2], ub[N // 2:]], axis=1)
    v_ref[...] = jnp.concatenate([v[:N // 2], v[N // 2:]], axis=1)


def _sc_body(table_hbm, fidx_hbm, m_hbm, s_hbm, s2_hbm,
             idx_all, rows0, rows1, outm0, outs0, outm1, outs1, s2acc,
             sem0, sem1, wsem0, wsem1):
    wid = lax.axis_index("s") * NC + lax.axis_index("c")

    pltpu.sync_copy(fidx_hbm.at[pl.ds(wid * IDX_ROWS, IDX_ROWS)], idx_all)

    off = (wid // (NW // B)) * N

    def add_off(r, carry):
        for g in range(8):
            sl = pl.ds(g * 16, 16)
            n = idx_all[r, sl]
            idx_all[r, sl] = (off + 2 * (n & (N // 2 - 1))
                              + lax.shift_right_logical(n, 12))
        return carry
    lax.fori_loop(0, IDX_ROWS, add_off, 0)

    zero = jnp.zeros((16,), jnp.float32)
    for g in range(O // 16):
        s2acc[0, pl.ds(g * 16, 16)] = zero

    def issue(chunk, rows_ref, sem):
        for g in range(GPC):
            pltpu.async_copy(
                table_hbm.at[idx_all.at[chunk * GPC + g]],
                rows_ref.at[pl.ds(g * 128, 128)],
                sem)

    def drain(rows_ref, sem):
        pltpu.make_async_copy(table_hbm.at[pl.ds(0, RPC)], rows_ref, sem).wait()

    def compute(chunk, rows_ref, outm, outs, wsem):
        mask_hi = jnp.full((16,), -65536, jnp.int32)

        rowb = ((wid // (NW // B)) * (N // 2) + (wid % (NW // B // 2)) * CPT
                + chunk * CHUNK)
        half = pl.ds(((wid % (NW // B)) // (NW // B // 2)) * O, O)

        @pl.when(chunk >= 2)
        def _():
            pltpu.make_async_copy(outm, m_hbm.at[pl.ds(rowb, CHUNK), half],
                                  wsem).wait()
            pltpu.make_async_copy(outs, s_hbm.at[pl.ds(rowb, CHUNK), half],
                                  wsem).wait()

        def bf16x2(w):
            a = lax.bitcast_convert_type(lax.shift_left(w, 16), jnp.float32)
            b = lax.bitcast_convert_type(w & mask_hi, jnp.float32)
            return a, b

        def center_body(c, carry):
            r0 = c * K
            for g2 in range(O // 32):
                col2 = pl.ds(g2 * 16, 16)
                va, vb = bf16x2(rows_ref[r0, col2])
                ma, sa, qa = va, va, va * va
                mb, sb, qb = vb, vb, vb * vb
                for r in range(1, K):
                    va, vb = bf16x2(rows_ref[r0 + r, col2])
                    ma = jnp.maximum(ma, va)
                    sa = sa + va
                    qa = qa + va * va
                    mb = jnp.maximum(mb, vb)
                    sb = sb + vb
                    qb = qb + vb * vb
                cola = pl.ds(g2 * 32, 16)
                colb = pl.ds(g2 * 32 + 16, 16)
                outm[c, cola] = ma
                outm[c, colb] = mb
                outs[c, cola] = sa
                outs[c, colb] = sb
                s2acc[0, cola] = s2acc[0, cola] + qa
                s2acc[0, colb] = s2acc[0, colb] + qb
            return carry
        lax.fori_loop(0, CHUNK, center_body, 0)
        pltpu.async_copy(outm, m_hbm.at[pl.ds(rowb, CHUNK), half], wsem)
        pltpu.async_copy(outs, s_hbm.at[pl.ds(rowb, CHUNK), half], wsem)

    issue(0, rows0, sem0)

    def outer(p, carry):
        a = p * 2
        drain(rows0, sem0)
        issue(a + 1, rows1, sem1)
        compute(a, rows0, outm0, outs0, wsem0)
        drain(rows1, sem1)

        @pl.when(p < NCHUNK // 2 - 1)
        def _():
            issue(a + 2, rows0, sem0)

        compute(a + 1, rows1, outm1, outs1, wsem1)
        return carry

    lax.fori_loop(0, NCHUNK // 2, outer, 0)
    for om, os, ws in ((outm0, outs0, wsem0), (outm1, outs1, wsem1)):
        pltpu.make_async_copy(om, m_hbm.at[pl.ds(0, CHUNK),
                                           pl.ds(0, O)], ws).wait()
        pltpu.make_async_copy(os, s_hbm.at[pl.ds(0, CHUNK),
                                           pl.ds(0, O)], ws).wait()
    pltpu.sync_copy(s2acc.at[0],
                    s2_hbm.at[wid // 2, pl.ds((wid % 2) * O, O)])


def _make_sc_gather():
    return pl.kernel(
        _sc_body,
        out_type=(
            jax.ShapeDtypeStruct((BN // 2, 2 * O), jnp.float32),
            jax.ShapeDtypeStruct((BN // 2, 2 * O), jnp.float32),
            jax.ShapeDtypeStruct((NW // 2, 2 * O), jnp.float32),
        ),
        mesh=plsc.VectorSubcoreMesh(core_axis_name="c", subcore_axis_name="s",
                                    num_cores=NC, num_subcores=NS),
        scratch_types=[
            pltpu.VMEM((IDX_ROWS, 128), jnp.int32),
            pltpu.VMEM((RPC, O // 2), jnp.int32),
            pltpu.VMEM((RPC, O // 2), jnp.int32),
            pltpu.VMEM((CHUNK, O), jnp.float32),
            pltpu.VMEM((CHUNK, O), jnp.float32),
            pltpu.VMEM((CHUNK, O), jnp.float32),
            pltpu.VMEM((CHUNK, O), jnp.float32),
            pltpu.VMEM((1, O), jnp.float32),
            pltpu.SemaphoreType.DMA,
            pltpu.SemaphoreType.DMA,
            pltpu.SemaphoreType.DMA,
            pltpu.SemaphoreType.DMA,
        ],
        compiler_params=pltpu.CompilerParams(use_tc_tiling_on_sc=False),
    )


def _norm_body(m_ref, s_ref, v_ref, s2_ref, o_ref):
    Sb = s_ref[...]
    vb = v_ref[...]
    Mb = m_ref[...]

    def fold(z):
        return z[:O] + z[O:]

    sS = fold(jnp.sum(Sb, axis=0))
    sv = fold(jnp.sum(vb, axis=0))
    sv2 = fold(jnp.sum(vb * vb, axis=0))
    cross = fold(jnp.sum(vb * Sb, axis=0))
    s2 = fold(jnp.sum(s2_ref[0], axis=0))
    cnt = float(N * K)
    mean = (sS + K * sv) / cnt
    eh2 = (s2 + 2.0 * cross + K * sv2) / cnt
    var = eh2 - mean * mean
    rstd = lax.rsqrt(var + EPS)
    mean2 = jnp.concatenate([mean, mean])
    rstd2 = jnp.concatenate([rstd, rstd])
    t = (Mb + vb - mean2[None, :]) * rstd2[None, :]
    t = jnp.where(t >= 0, t, SLOPE * t)
    o_ref[0] = jnp.concatenate([t[:, :O].T, t[:, O:].T], axis=1)


def kernel(x, idx, W0):
    x = x.astype(jnp.float32)
    idx32 = idx.astype(jnp.int32)
    A = W0[:, :C]
    Qm = W0[:, C:] - A
    perm = [b + (i // 2) + 16 * (i % 2) for b in (0, 32) for i in range(32)]
    wa = A.T[:, perm]
    wq = Qm.T

    u, v = pl.pallas_call(
        _mm_body,
        grid=(B,),
        in_specs=[
            pl.BlockSpec((1, C, N), lambda b: (b, 0, 0)),
            pl.BlockSpec((C, O), lambda b: (0, 0)),
            pl.BlockSpec((C, O), lambda b: (0, 0)),
        ],
        out_specs=[
            pl.BlockSpec((N // 2, 2 * O), lambda b: (b, 0)),
            pl.BlockSpec((N // 2, 2 * O), lambda b: (b, 0)),
        ],
        out_shape=[
            jax.ShapeDtypeStruct((BN // 2, 2 * O), jnp.bfloat16),
            jax.ShapeDtypeStruct((BN // 2, 2 * O), jnp.float32),
        ],
    )(x, wa, wq)

    fidx = idx32.reshape(BN * K // 128, 128)
    table = lax.bitcast_convert_type(
        u.reshape(BN // 2, O, 2), jnp.int32).reshape(BN, O // 2)

    Mg, Sg, S2p = _make_sc_gather()(table, fidx)

    out = pl.pallas_call(
        _norm_body,
        grid=(B,),
        in_specs=[
            pl.BlockSpec((N // 2, 2 * O), lambda b: (b, 0)),
            pl.BlockSpec((N // 2, 2 * O), lambda b: (b, 0)),
            pl.BlockSpec((N // 2, 2 * O), lambda b: (b, 0)),
            pl.BlockSpec((1, NW // B // 2, 2 * O), lambda b: (b, 0, 0)),
        ],
        out_specs=pl.BlockSpec((1, O, N), lambda b: (b, 0, 0)),
        out_shape=jax.ShapeDtypeStruct((B, O, N), jnp.float32),
    )(Mg, Sg, v, S2p.reshape(B, NW // B // 2, 2 * O))
    return out


# speedup vs baseline: 1.2666x; 1.2173x over previous
import jax
import jax.numpy as jnp
from jax import lax
from jax.experimental import pallas as pl
from jax.experimental.pallas import tpu as pltpu
from jax.experimental.pallas import tpu_sc as plsc

B, C, N, K = 4, 64, 8192, 16
O = 64
EPS = 1e-5
SLOPE = 0.2
BN = B * N

NC, NS = 2, 16
NW = NC * NS
CPT = BN // NW
CHUNK = 32
NCHUNK = CPT // CHUNK
RPC = CHUNK * K
GPC = RPC // 128
IDX_ROWS = CPT * K // 128


def _mm_body(x_ref, wa_ref, wq_ref, u_ref, v_ref):
    xb = x_ref[0]
    dn = (((0,), (0,)), ((), ()))
    u = lax.dot_general(xb, wa_ref[...], dn,
                        preferred_element_type=jnp.float32)
    v = lax.dot_general(xb, wq_ref[...], dn,
                        preferred_element_type=jnp.float32)
    u_ref[...] = jnp.concatenate([u[:N // 2], u[N // 2:]], axis=1)
    v_ref[...] = jnp.concatenate([v[:N // 2], v[N // 2:]], axis=1)


def _sc_body(table_hbm, fidx_hbm, m_hbm, s_hbm, s2_hbm,
             idx_all, rows0, rows1, outm0, outs0, outm1, outs1, s2acc,
             sem0, sem1, wsem0, wsem1):
    wid = lax.axis_index("s") * NC + lax.axis_index("c")

    pltpu.sync_copy(fidx_hbm.at[pl.ds(wid * IDX_ROWS, IDX_ROWS)], idx_all)

    off = (wid // (NW // B)) * N

    def add_off(r, carry):
        for g in range(8):
            sl = pl.ds(g * 16, 16)
            n = idx_all[r, sl]
            idx_all[r, sl] = (off + 2 * (n & (N // 2 - 1))
                              + lax.shift_right_logical(n, 12))
        return carry
    lax.fori_loop(0, IDX_ROWS, add_off, 0)

    def issue(chunk, rows_ref, sem):
        for g in range(GPC):
            pltpu.async_copy(
                table_hbm.at[idx_all.at[chunk * GPC + g]],
                rows_ref.at[pl.ds(g * 128, 128)],
                sem)

    def drain(rows_ref, sem):
        pltpu.make_async_copy(table_hbm.at[pl.ds(0, RPC)], rows_ref, sem).wait()

    def compute(chunk, rows_ref, outm, outs, wsem, s2state):
        rowb = ((wid // (NW // B)) * (N // 2) + (wid % (NW // B // 2)) * CPT
                + chunk * CHUNK)
        half = pl.ds(((wid % (NW // B)) // (NW // B // 2)) * O, O)

        @pl.when(chunk >= 2)
        def _():
            pltpu.make_async_copy(outm, m_hbm.at[pl.ds(rowb, CHUNK), half],
                                  wsem).wait()
            pltpu.make_async_copy(outs, s_hbm.at[pl.ds(rowb, CHUNK), half],
                                  wsem).wait()

        def center_body(c, s2):
            r0 = c * K
            new = []
            for g in range(O // 16):
                col = pl.ds(g * 16, 16)
                val = rows_ref[r0, col]
                m = val
                s = val
                q = val * val
                for r in range(1, K):
                    val = rows_ref[r0 + r, col]
                    m = jnp.maximum(m, val)
                    s = s + val
                    q = q + val * val
                outm[c, col] = m
                outs[c, col] = s
                new.append(s2[g] + q)
            return tuple(new)
        s2state = plsc.parallel_loop(0, CHUNK, step=1, unroll=2,
                                     carry=s2state)(center_body)
        pltpu.async_copy(outm, m_hbm.at[pl.ds(rowb, CHUNK), half], wsem)
        pltpu.async_copy(outs, s_hbm.at[pl.ds(rowb, CHUNK), half], wsem)
        return s2state

    issue(0, rows0, sem0)

    def outer(p, s2state):
        a = p * 2
        drain(rows0, sem0)
        issue(a + 1, rows1, sem1)
        s2state = compute(a, rows0, outm0, outs0, wsem0, s2state)
        drain(rows1, sem1)

        @pl.when(p < NCHUNK // 2 - 1)
        def _():
            issue(a + 2, rows0, sem0)

        return compute(a + 1, rows1, outm1, outs1, wsem1, s2state)

    zero = jnp.zeros((16,), jnp.float32)
    s2fin = lax.fori_loop(0, NCHUNK // 2, outer,
                          tuple(zero for _ in range(O // 16)))
    for g in range(O // 16):
        s2acc[0, pl.ds(g * 16, 16)] = s2fin[g]
    for om, os, ws in ((outm0, outs0, wsem0), (outm1, outs1, wsem1)):
        pltpu.make_async_copy(om, m_hbm.at[pl.ds(0, CHUNK),
                                           pl.ds(0, O)], ws).wait()
        pltpu.make_async_copy(os, s_hbm.at[pl.ds(0, CHUNK),
                                           pl.ds(0, O)], ws).wait()
    pltpu.sync_copy(s2acc.at[0],
                    s2_hbm.at[wid // 2, pl.ds((wid % 2) * O, O)])


def _make_sc_gather():
    return pl.kernel(
        _sc_body,
        out_type=(
            jax.ShapeDtypeStruct((BN // 2, 2 * O), jnp.float32),
            jax.ShapeDtypeStruct((BN // 2, 2 * O), jnp.float32),
            jax.ShapeDtypeStruct((NW // 2, 2 * O), jnp.float32),
        ),
        mesh=plsc.VectorSubcoreMesh(core_axis_name="c", subcore_axis_name="s",
                                    num_cores=NC, num_subcores=NS),
        scratch_types=[
            pltpu.VMEM((IDX_ROWS, 128), jnp.int32),
            pltpu.VMEM((RPC, O), jnp.float32),
            pltpu.VMEM((RPC, O), jnp.float32),
            pltpu.VMEM((CHUNK, O), jnp.float32),
            pltpu.VMEM((CHUNK, O), jnp.float32),
            pltpu.VMEM((CHUNK, O), jnp.float32),
            pltpu.VMEM((CHUNK, O), jnp.float32),
            pltpu.VMEM((1, O), jnp.float32),
            pltpu.SemaphoreType.DMA,
            pltpu.SemaphoreType.DMA,
            pltpu.SemaphoreType.DMA,
            pltpu.SemaphoreType.DMA,
        ],
        compiler_params=pltpu.CompilerParams(use_tc_tiling_on_sc=False),
    )


def _norm_body(m_ref, s_ref, v_ref, s2_ref, o_ref):
    Sb = s_ref[...]
    vb = v_ref[...]
    Mb = m_ref[...]

    def fold(z):
        return z[:O] + z[O:]

    sS = fold(jnp.sum(Sb, axis=0))
    sv = fold(jnp.sum(vb, axis=0))
    sv2 = fold(jnp.sum(vb * vb, axis=0))
    cross = fold(jnp.sum(vb * Sb, axis=0))
    s2 = fold(jnp.sum(s2_ref[0], axis=0))
    cnt = float(N * K)
    mean = (sS + K * sv) / cnt
    eh2 = (s2 + 2.0 * cross + K * sv2) / cnt
    var = eh2 - mean * mean
    rstd = lax.rsqrt(var + EPS)
    mean2 = jnp.concatenate([mean, mean])
    rstd2 = jnp.concatenate([rstd, rstd])
    t = (Mb + vb - mean2[None, :]) * rstd2[None, :]
    t = jnp.where(t >= 0, t, SLOPE * t)
    o_ref[0] = jnp.concatenate([t[:, :O].T, t[:, O:].T], axis=1)


def kernel(x, idx, W0):
    x = x.astype(jnp.float32)
    idx32 = idx.astype(jnp.int32)
    A = W0[:, :C]
    Qm = W0[:, C:] - A
    wa = A.T
    wq = Qm.T

    u, v = pl.pallas_call(
        _mm_body,
        grid=(B,),
        in_specs=[
            pl.BlockSpec((1, C, N), lambda b: (b, 0, 0)),
            pl.BlockSpec((C, O), lambda b: (0, 0)),
            pl.BlockSpec((C, O), lambda b: (0, 0)),
        ],
        out_specs=[
            pl.BlockSpec((N // 2, 2 * O), lambda b: (b, 0)),
            pl.BlockSpec((N // 2, 2 * O), lambda b: (b, 0)),
        ],
        out_shape=[
            jax.ShapeDtypeStruct((BN // 2, 2 * O), jnp.float32),
            jax.ShapeDtypeStruct((BN // 2, 2 * O), jnp.float32),
        ],
    )(x, wa, wq)

    fidx = idx32.reshape(BN * K // 128, 128)
    table = u.reshape(BN, O)

    Mg, Sg, S2p = _make_sc_gather()(table, fidx)

    out = pl.pallas_call(
        _norm_body,
        grid=(B,),
        in_specs=[
            pl.BlockSpec((N // 2, 2 * O), lambda b: (b, 0)),
            pl.BlockSpec((N // 2, 2 * O), lambda b: (b, 0)),
            pl.BlockSpec((N // 2, 2 * O), lambda b: (b, 0)),
            pl.BlockSpec((1, NW // B // 2, 2 * O), lambda b: (b, 0, 0)),
        ],
        out_specs=pl.BlockSpec((1, O, N), lambda b: (b, 0, 0)),
        out_shape=jax.ShapeDtypeStruct((B, O, N), jnp.float32),
    )(Mg, Sg, v, S2p.reshape(B, NW // B // 2, 2 * O))
    return out
